# baseline (speedup 1.0000x reference)
import jax
import jax.numpy as jnp
from jax.experimental import pallas as pl
from jax.experimental.pallas import tpu as pltpu

_R, _C = 2048, 8192
_BR = 64


def _minmax_body(x_ref, mn_ref, mx_ref, acc_ref):
    i = pl.program_id(0)
    n = pl.num_programs(0)
    bmin = jnp.min(x_ref[...])
    bmax = jnp.max(x_ref[...])

    @pl.when(i == 0)
    def _():
        acc_ref[0] = bmin
        acc_ref[1] = bmax

    @pl.when(i > 0)
    def _():
        acc_ref[0] = jnp.minimum(acc_ref[0], bmin)
        acc_ref[1] = jnp.maximum(acc_ref[1], bmax)

    @pl.when(i == n - 1)
    def _():
        mn_ref[0] = acc_ref[0]
        mx_ref[0] = acc_ref[1]


def _quant_body(s_ref, x_ref, o_ref):
    mn = s_ref[0]
    sc = s_ref[1]
    o_ref[...] = jnp.round((x_ref[...] - mn) / sc) * sc + mn


def kernel(tensor, bits):
    x = tensor.reshape(_R, _C)

    mn, mx = pl.pallas_call(
        _minmax_body,
        grid=(_R // _BR,),
        in_specs=[pl.BlockSpec((_BR, _C), lambda i: (i, 0))],
        out_specs=[
            pl.BlockSpec(memory_space=pltpu.SMEM),
            pl.BlockSpec(memory_space=pltpu.SMEM),
        ],
        out_shape=[
            jax.ShapeDtypeStruct((1,), jnp.float32),
            jax.ShapeDtypeStruct((1,), jnp.float32),
        ],
        scratch_shapes=[pltpu.SMEM((2,), jnp.float32)],
    )(x)

    min_val = mn[0]
    scale = (mx[0] - min_val) / (2 ** bits - 1)
    s = jnp.stack([min_val, scale])

    y = pl.pallas_call(
        _quant_body,
        grid=(_R // _BR,),
        in_specs=[
            pl.BlockSpec(memory_space=pltpu.SMEM),
            pl.BlockSpec((_BR, _C), lambda i: (i, 0)),
        ],
        out_specs=pl.BlockSpec((_BR, _C), lambda i: (i, 0)),
        out_shape=jax.ShapeDtypeStruct((_R, _C), jnp.float32),
    )(s, x)

    return y.reshape(tensor.shape)


# speedup vs baseline: 1.7572x; 1.7572x over previous
import jax
import jax.numpy as jnp
from jax.experimental import pallas as pl
from jax.experimental.pallas import tpu as pltpu

_N = 16777216
_BN = 1 << 19


def _minmax_body(x_ref, mn_ref, mx_ref, acc_ref):
    i = pl.program_id(0)
    n = pl.num_programs(0)
    bmin = jnp.min(x_ref[...])
    bmax = jnp.max(x_ref[...])

    @pl.when(i == 0)
    def _():
        acc_ref[0] = bmin
        acc_ref[1] = bmax

    @pl.when(i > 0)
    def _():
        acc_ref[0] = jnp.minimum(acc_ref[0], bmin)
        acc_ref[1] = jnp.maximum(acc_ref[1], bmax)

    @pl.when(i == n - 1)
    def _():
        mn_ref[0] = acc_ref[0]
        mx_ref[0] = acc_ref[1]


def _quant_body(s_ref, x_ref, o_ref):
    mn = s_ref[0]
    sc = s_ref[1]
    o_ref[...] = jnp.round((x_ref[...] - mn) / sc) * sc + mn


def kernel(tensor, bits):
    mn, mx = pl.pallas_call(
        _minmax_body,
        grid=(_N // _BN,),
        in_specs=[pl.BlockSpec((_BN,), lambda i: (i,))],
        out_specs=[
            pl.BlockSpec(memory_space=pltpu.SMEM),
            pl.BlockSpec(memory_space=pltpu.SMEM),
        ],
        out_shape=[
            jax.ShapeDtypeStruct((1,), jnp.float32),
            jax.ShapeDtypeStruct((1,), jnp.float32),
        ],
        scratch_shapes=[pltpu.SMEM((2,), jnp.float32)],
    )(tensor)

    min_val = mn[0]
    scale = (mx[0] - min_val) / (2 ** bits - 1)
    s = jnp.stack([min_val, scale])

    y = pl.pallas_call(
        _quant_body,
        grid=(_N // _BN,),
        in_specs=[
            pl.BlockSpec(memory_space=pltpu.SMEM),
            pl.BlockSpec((_BN,), lambda i: (i,)),
        ],
        out_specs=pl.BlockSpec((_BN,), lambda i: (i,)),
        out_shape=jax.ShapeDtypeStruct((_N,), jnp.float32),
    )(s, tensor)

    return y


# speedup vs baseline: 3.0860x; 1.7562x over previous
import jax
import jax.numpy as jnp
from jax.experimental import pallas as pl
from jax.experimental.pallas import tpu as pltpu

_N = 16777216
_R, _C = _N // 128, 128
_BR = 1 << 14


def _minmax_body(x_ref, mn_ref, mx_ref, acc_ref):
    i = pl.program_id(0)
    n = pl.num_programs(0)
    bmin = jnp.min(x_ref[...])
    bmax = jnp.max(x_ref[...])

    @pl.when(i == 0)
    def _():
        acc_ref[0] = bmin
        acc_ref[1] = bmax

    @pl.when(i > 0)
    def _():
        acc_ref[0] = jnp.minimum(acc_ref[0], bmin)
        acc_ref[1] = jnp.maximum(acc_ref[1], bmax)

    @pl.when(i == n - 1)
    def _():
        mn_ref[0] = acc_ref[0]
        mx_ref[0] = acc_ref[1]


def _quant_body(s_ref, x_ref, o_ref):
    mn = s_ref[0]
    sc = s_ref[1]
    inv = s_ref[2]
    o_ref[...] = jnp.round((x_ref[...] - mn) * inv) * sc + mn


def kernel(tensor, bits):
    x = tensor.reshape(_R, _C)

    mn, mx = pl.pallas_call(
        _minmax_body,
        grid=(_R // _BR,),
        in_specs=[pl.BlockSpec((_BR, _C), lambda i: (i, 0))],
        out_specs=[
            pl.BlockSpec(memory_space=pltpu.SMEM),
            pl.BlockSpec(memory_space=pltpu.SMEM),
        ],
        out_shape=[
            jax.ShapeDtypeStruct((1,), jnp.float32),
            jax.ShapeDtypeStruct((1,), jnp.float32),
        ],
        scratch_shapes=[pltpu.SMEM((2,), jnp.float32)],
    )(x)

    min_val = mn[0]
    scale = (mx[0] - min_val) / (2 ** bits - 1)
    s = jnp.stack([min_val, scale, 1.0 / scale])

    y = pl.pallas_call(
        _quant_body,
        grid=(_R // _BR,),
        in_specs=[
            pl.BlockSpec(memory_space=pltpu.SMEM),
            pl.BlockSpec((_BR, _C), lambda i: (i, 0)),
        ],
        out_specs=pl.BlockSpec((_BR, _C), lambda i: (i, 0)),
        out_shape=jax.ShapeDtypeStruct((_R, _C), jnp.float32),
    )(s, x)

    return y.reshape(tensor.shape)


# speedup vs baseline: 3.5893x; 1.1631x over previous
import jax
import jax.numpy as jnp
from jax.experimental import pallas as pl
from jax.experimental.pallas import tpu as pltpu

_N = 16777216
_R, _C = _N // 128, 128
_G = 32
_CR = _R // _G
_RESCH = 22
_SLOTS = _RESCH + 3


def _slot(j):
    if isinstance(j, int):
        return j if j < _RESCH else _RESCH + (j % 3)
    return jnp.where(j < _RESCH, j, _RESCH + (j % 3))


def _copy(x_hbm, buf, sems, j):
    return pltpu.make_async_copy(
        x_hbm.at[pl.ds(j * _CR, _CR), :],
        buf.at[pl.ds(_slot(j) * _CR, _CR), :],
        sems.at[_slot(j)],
    )


def _body(denom_ref, x_hbm, o_ref, buf, acc, sems):
    i = pl.program_id(0)

    @pl.when(i == 0)
    def _():
        for j in range(_RESCH + 3):
            _copy(x_hbm, buf, sems, j).start()

    @pl.when(i < _G)
    def _():
        _copy(x_hbm, buf, sems, i).wait()
        chunk = buf[pl.ds(_slot(i) * _CR, _CR), :]
        bmin = jnp.min(chunk)
        bmax = jnp.max(chunk)

        @pl.when(i == 0)
        def _():
            acc[0] = bmin
            acc[1] = bmax

        @pl.when(i > 0)
        def _():
            acc[0] = jnp.minimum(acc[0], bmin)
            acc[1] = jnp.maximum(acc[1], bmax)

        if _RESCH + 3 < _G:

            @pl.when(jnp.logical_and(i >= _RESCH, i + 3 < _G))
            def _():
                _copy(x_hbm, buf, sems, i + 3).start()

    @pl.when(i >= _G)
    def _():
        j = i - _G

        @pl.when(j == 0)
        def _():
            for jj in range(_RESCH, min(_RESCH + 3, _G)):
                _copy(x_hbm, buf, sems, jj).start()

        @pl.when(j >= _RESCH)
        def _():
            _copy(x_hbm, buf, sems, j).wait()

        mn = acc[0]
        sc = (acc[1] - mn) / denom_ref[0]
        inv = 1.0 / sc
        chunk = buf[pl.ds(_slot(j) * _CR, _CR), :]
        o_ref[...] = jnp.round((chunk - mn) * inv) * sc + mn

        @pl.when(jnp.logical_and(j >= _RESCH, j + 3 < _G))
        def _():
            _copy(x_hbm, buf, sems, j + 3).start()


def kernel(tensor, bits):
    x = tensor.reshape(_R, _C)
    denom = jnp.asarray((2 ** bits) - 1, dtype=jnp.float32).reshape(1)

    y = pl.pallas_call(
        _body,
        grid=(2 * _G,),
        in_specs=[
            pl.BlockSpec(memory_space=pltpu.SMEM),
            pl.BlockSpec(memory_space=pl.ANY),
        ],
        out_specs=pl.BlockSpec(
            (_CR, _C), lambda i: (jnp.where(i < _G, 0, i - _G), 0)
        ),
        out_shape=jax.ShapeDtypeStruct((_R, _C), jnp.float32),
        scratch_shapes=[
            pltpu.VMEM((_SLOTS * _CR, _C), jnp.float32),
            pltpu.SMEM((2,), jnp.float32),
            pltpu.SemaphoreType.DMA((_SLOTS,)),
        ],
    )(denom, x)

    return y.reshape(tensor.shape)


# speedup vs baseline: 3.8459x; 1.0715x over previous
import jax
import jax.numpy as jnp
from jax.experimental import pallas as pl
from jax.experimental.pallas import tpu as pltpu

_N = 16777216
_R, _C = _N // 128, 128
_G = 32
_CR = _R // _G
_RESCH = 22
_SLOTS = _RESCH + 3
_SUB = 128


def _slot(j):
    if isinstance(j, int):
        return j if j < _RESCH else _RESCH + (j % 3)
    return jnp.where(j < _RESCH, j, _RESCH + (j % 3))


def _copy(x_hbm, buf, sems, j):
    return pltpu.make_async_copy(
        x_hbm.at[pl.ds(j * _CR, _CR), :],
        buf.at[pl.ds(_slot(j) * _CR, _CR), :],
        sems.at[_slot(j)],
    )


def _body(denom_ref, x_hbm, o_ref, buf, acc, sems):
    i = pl.program_id(0)

    @pl.when(i == 0)
    def _():
        for j in range(_RESCH + 3):
            _copy(x_hbm, buf, sems, j).start()

    @pl.when(i < _G)
    def _():
        _copy(x_hbm, buf, sems, i).wait()
        base = _slot(i) * _CR

        def _red(k, carry):
            cmn, cmx = carry
            v = buf[pl.ds(base + k * _SUB, _SUB), :]
            return jnp.minimum(cmn, v), jnp.maximum(cmx, v)

        v0 = buf[pl.ds(base, _SUB), :]
        cmn, cmx = jax.lax.fori_loop(1, _CR // _SUB, _red, (v0, v0))
        bmin = jnp.min(cmn)
        bmax = jnp.max(cmx)

        @pl.when(i == 0)
        def _():
            acc[0] = bmin
            acc[1] = bmax

        @pl.when(i > 0)
        def _():
            acc[0] = jnp.minimum(acc[0], bmin)
            acc[1] = jnp.maximum(acc[1], bmax)

        if _RESCH + 3 < _G:

            @pl.when(jnp.logical_and(i >= _RESCH, i + 3 < _G))
            def _():
                _copy(x_hbm, buf, sems, i + 3).start()

    @pl.when(i >= _G)
    def _():
        j = i - _G

        @pl.when(j == 0)
        def _():
            for jj in range(_RESCH, min(_RESCH + 3, _G)):
                _copy(x_hbm, buf, sems, jj).start()

        @pl.when(j >= _RESCH)
        def _():
            _copy(x_hbm, buf, sems, j).wait()

        mn = acc[0]
        sc = (acc[1] - mn) / denom_ref[0]
        inv = 1.0 / sc
        base = _slot(j) * _CR

        def _quant(k, carry):
            v = buf[pl.ds(base + k * _SUB, _SUB), :]
            o_ref[pl.ds(k * _SUB, _SUB), :] = (
                jnp.round((v - mn) * inv) * sc + mn
            )
            return carry

        jax.lax.fori_loop(0, _CR // _SUB, _quant, 0)

        @pl.when(jnp.logical_and(j >= _RESCH, j + 3 < _G))
        def _():
            _copy(x_hbm, buf, sems, j + 3).start()


def kernel(tensor, bits):
    x = tensor.reshape(_R, _C)
    denom = jnp.asarray((2 ** bits) - 1, dtype=jnp.float32).reshape(1)

    y = pl.pallas_call(
        _body,
        grid=(2 * _G,),
        in_specs=[
            pl.BlockSpec(memory_space=pltpu.SMEM),
            pl.BlockSpec(memory_space=pl.ANY),
        ],
        out_specs=pl.BlockSpec(
            (_CR, _C), lambda i: (jnp.where(i < _G, 0, i - _G), 0)
        ),
        out_shape=jax.ShapeDtypeStruct((_R, _C), jnp.float32),
        scratch_shapes=[
            pltpu.VMEM((_SLOTS * _CR, _C), jnp.float32),
            pltpu.SMEM((2,), jnp.float32),
            pltpu.SemaphoreType.DMA((_SLOTS,)),
        ],
    )(denom, x)

    return y.reshape(tensor.shape)
